# Initial kernel scaffold; baseline (speedup 1.0000x reference)
import functools

import jax
import jax.numpy as jnp
from jax import lax
from jax.experimental import pallas as pl
from jax.experimental.pallas import tpu as pltpu
from jax.experimental.pallas import tpu_sc as plsc

_VOCAB = 1000000
_HIDDEN = 32
_BATCH = 16384
_HIST = 50

_NC = 2
_NS = 16
_NW = _NC * _NS

_TOTAL = _BATCH * _HIST
_PER_W = _TOTAL // _NW
_IDX_MINOR = 128
_SUB = 8
_CHUNK = _SUB * _IDX_MINOR
_OUTER = _PER_W // _CHUNK


@functools.partial(
    pl.kernel,
    out_type=jax.ShapeDtypeStruct((_NW, _OUTER, _CHUNK, _HIDDEN), jnp.float32),
    mesh=plsc.VectorSubcoreMesh(
        core_axis_name="c", subcore_axis_name="s",
        num_cores=_NC, num_subcores=_NS,
    ),
    scratch_types=[
        pltpu.VMEM((_SUB, _IDX_MINOR), jnp.int32),
        pltpu.VMEM((_CHUNK, _HIDDEN), jnp.float32),
        pltpu.SemaphoreType.DMA,
    ],
)
def _gather(table, xr, out, idx_v, rows_v, gsem):
    wid = lax.axis_index("s") * _NC + lax.axis_index("c")

    def chunk(j, carry):
        pltpu.sync_copy(xr.at[wid, j], idx_v)
        futs = [
            pltpu.async_copy(
                table.at[idx_v.at[t]],
                rows_v.at[pl.ds(t * _IDX_MINOR, _IDX_MINOR)],
                gsem,
            )
            for t in range(_SUB)
        ]
        for f in futs:
            f.wait()
        pltpu.sync_copy(rows_v, out.at[wid, j])
        return carry

    lax.fori_loop(0, _OUTER, chunk, 0)


def kernel(x, weight):
    xr = x.astype(jnp.int32).reshape(_NW, _OUTER, _SUB, _IDX_MINOR)
    out = _gather(weight, xr)
    return out.reshape(_BATCH, _HIST, _HIDDEN)


# speedup vs baseline: 1.1068x; 1.1068x over previous
import functools

import jax
import jax.numpy as jnp
from jax import lax
from jax.experimental import pallas as pl
from jax.experimental.pallas import tpu as pltpu
from jax.experimental.pallas import tpu_sc as plsc

_VOCAB = 1000000
_HIDDEN = 32
_BATCH = 16384
_HIST = 50

_NC = 2
_NS = 16
_NW = _NC * _NS

_TOTAL = _BATCH * _HIST
_PER_W = _TOTAL // _NW
_IDX_MINOR = 128
_SUB = 8
_CHUNK = _SUB * _IDX_MINOR
_OUTER = _PER_W // _CHUNK


@functools.partial(
    pl.kernel,
    out_type=jax.ShapeDtypeStruct((_NW, _OUTER, _CHUNK, _HIDDEN), jnp.float32),
    mesh=plsc.VectorSubcoreMesh(
        core_axis_name="c", subcore_axis_name="s",
        num_cores=_NC, num_subcores=_NS,
    ),
    scratch_types=[
        pltpu.VMEM((_SUB, _IDX_MINOR), jnp.int32),
        pltpu.VMEM((_CHUNK, _HIDDEN), jnp.float32),
        pltpu.SemaphoreType.DMA,
    ],
    compiler_params=pltpu.CompilerParams(use_tc_tiling_on_sc=False),
)
def _gather(table, xr, out, idx_v, rows_v, gsem):
    wid = lax.axis_index("s") * _NC + lax.axis_index("c")

    def chunk(j, carry):
        pltpu.sync_copy(xr.at[wid, j], idx_v)
        futs = [
            pltpu.async_copy(
                table.at[idx_v.at[t]],
                rows_v.at[pl.ds(t * _IDX_MINOR, _IDX_MINOR)],
                gsem,
            )
            for t in range(_SUB)
        ]
        for f in futs:
            f.wait()
        pltpu.sync_copy(rows_v, out.at[wid, j])
        return carry

    lax.fori_loop(0, _OUTER, chunk, 0)


def kernel(x, weight):
    xr = x.astype(jnp.int32).reshape(_NW, _OUTER, _SUB, _IDX_MINOR)
    out = _gather(weight, xr)
    return out.reshape(_BATCH, _HIST, _HIDDEN)


# speedup vs baseline: 1.1704x; 1.0575x over previous
import functools

import jax
import jax.numpy as jnp
from jax import lax
from jax.experimental import pallas as pl
from jax.experimental.pallas import tpu as pltpu
from jax.experimental.pallas import tpu_sc as plsc

_VOCAB = 1000000
_HIDDEN = 32
_BATCH = 16384
_HIST = 50

_NC = 2
_NS = 16
_NW = _NC * _NS

_TOTAL = _BATCH * _HIST
_PER_W = _TOTAL // _NW
_IDX_MINOR = 128
_SUB = 10
_CHUNK = _SUB * _IDX_MINOR
_OUTER = _PER_W // _CHUNK
_NPAIR = _OUTER // 2
_IDX_ROWS = _PER_W // _IDX_MINOR


@functools.partial(
    pl.kernel,
    out_type=jax.ShapeDtypeStruct((_NW, _OUTER, _CHUNK, _HIDDEN), jnp.float32),
    mesh=plsc.VectorSubcoreMesh(
        core_axis_name="c", subcore_axis_name="s",
        num_cores=_NC, num_subcores=_NS,
    ),
    scratch_types=[
        pltpu.VMEM((_IDX_ROWS, _IDX_MINOR), jnp.int32),
        pltpu.VMEM((_CHUNK, _HIDDEN), jnp.float32),
        pltpu.VMEM((_CHUNK, _HIDDEN), jnp.float32),
        pltpu.SemaphoreType.DMA,
        pltpu.SemaphoreType.DMA,
    ],
    compiler_params=pltpu.CompilerParams(use_tc_tiling_on_sc=False),
)
def _gather(table, xr, out, idx_v, buf0, buf1, gs0, gs1):
    wid = lax.axis_index("s") * _NC + lax.axis_index("c")

    def fire(j, buf, sem):
        for t in range(_SUB):
            pltpu.async_copy(
                table.at[idx_v.at[j * _SUB + t]],
                buf.at[pl.ds(t * _IDX_MINOR, _IDX_MINOR)],
                sem,
            )

    def drain(buf, sem):
        pltpu.make_async_copy(table.at[pl.ds(0, _CHUNK)], buf, sem).wait()

    pltpu.sync_copy(xr.at[wid], idx_v)
    fire(0, buf0, gs0)

    def body(i, carry):
        j0 = 2 * i
        fire(j0 + 1, buf1, gs1)
        drain(buf0, gs0)
        pltpu.sync_copy(buf0, out.at[wid, j0])
        fire(j0 + 2, buf0, gs0)
        drain(buf1, gs1)
        pltpu.sync_copy(buf1, out.at[wid, j0 + 1])
        return carry

    lax.fori_loop(0, _NPAIR - 1, body, 0)

    fire(_OUTER - 1, buf1, gs1)
    drain(buf0, gs0)
    pltpu.sync_copy(buf0, out.at[wid, _OUTER - 2])
    drain(buf1, gs1)
    pltpu.sync_copy(buf1, out.at[wid, _OUTER - 1])


def kernel(x, weight):
    xr = x.astype(jnp.int32).reshape(_NW, _IDX_ROWS, _IDX_MINOR)
    out = _gather(weight, xr)
    return out.reshape(_BATCH, _HIST, _HIDDEN)


# speedup vs baseline: 1.4955x; 1.2778x over previous
import functools

import jax
import jax.numpy as jnp
from jax import lax
from jax.experimental import pallas as pl
from jax.experimental.pallas import tpu as pltpu
from jax.experimental.pallas import tpu_sc as plsc

_VOCAB = 1000000
_HIDDEN = 32
_BATCH = 16384
_HIST = 50

_NC = 2
_NS = 16
_NW = _NC * _NS

_BG = _BATCH // 128
_BG_PER_W = _BG // _NW
_UNITS = _BG_PER_W * _HIST
_NBUF = 4
_NGRP = _UNITS // _NBUF


@functools.partial(
    pl.kernel,
    out_type=jax.ShapeDtypeStruct((_HIST, 4, _BG, 8, 128), jnp.float32),
    mesh=plsc.VectorSubcoreMesh(
        core_axis_name="c", subcore_axis_name="s",
        num_cores=_NC, num_subcores=_NS,
    ),
    scratch_types=[
        pltpu.VMEM((_BG_PER_W, _HIST, 128), jnp.int32),
        [pltpu.VMEM((128, _HIDDEN), jnp.float32) for _ in range(_NBUF)],
        [pltpu.VMEM((_HIDDEN, 128), jnp.float32) for _ in range(_NBUF)],
        [pltpu.SemaphoreType.DMA for _ in range(_NBUF)],
        [pltpu.SemaphoreType.DMA for _ in range(_NBUF)],
    ],
    compiler_params=pltpu.CompilerParams(
        use_tc_tiling_on_sc=False, needs_layout_passes=False
    ),
)
def _gather(table, idx3, out5, idxv, rows, tps, gs, os):
    wid = lax.axis_index("s") * _NC + lax.axis_index("c")
    pltpu.sync_copy(idx3.at[pl.ds(wid * _BG_PER_W, _BG_PER_W)], idxv)

    iota = lax.iota(jnp.int32, 16)
    bsvecs = [iota + 16 * j for j in range(8)]

    def fire(u, row, sem):
        bgl = u // _HIST
        t = u % _HIST
        pltpu.async_copy(table.at[idxv.at[bgl, t]], row, sem)

    def drain_gather(row, sem):
        pltpu.make_async_copy(table.at[pl.ds(0, 128)], row, sem).wait()

    def transpose(row, tbuf):
        def dbody(d, carry):
            dvec = jnp.full((16,), d, jnp.int32)
            for j in range(8):
                v = plsc.load_gather(row, [bsvecs[j], dvec])
                tbuf[d, pl.ds(16 * j, 16)] = v
            return carry

        lax.fori_loop(0, _HIDDEN, dbody, 0)

    def write_out(u, tbuf, sem):
        bg = wid * _BG_PER_W + u // _HIST
        t = u % _HIST
        for dg in range(4):
            pltpu.async_copy(
                tbuf.at[pl.ds(dg * 8, 8)], out5.at[t, dg, bg], sem
            )

    def drain_out(tbuf, sem):
        for dg in range(4):
            pltpu.make_async_copy(
                out5.at[0, 0, 0], tbuf.at[pl.ds(dg * 8, 8)], sem
            ).wait()

    for b in range(_NBUF):
        fire(b, rows[b], gs[b])

    def body(i, carry):
        for b in range(_NBUF):
            u = _NBUF * i + b
            drain_gather(rows[b], gs[b])

            @pl.when(i > 0)
            def _():
                drain_out(tps[b], os[b])

            transpose(rows[b], tps[b])
            write_out(u, tps[b], os[b])
            fire(u + _NBUF, rows[b], gs[b])
        return carry

    lax.fori_loop(0, _NGRP - 1, body, 0)

    for b in range(_NBUF):
        u = _NBUF * (_NGRP - 1) + b
        drain_gather(rows[b], gs[b])
        drain_out(tps[b], os[b])
        transpose(rows[b], tps[b])
        write_out(u, tps[b], os[b])
    for b in range(_NBUF):
        drain_out(tps[b], os[b])


def kernel(x, weight):
    xi = x.astype(jnp.int32)
    idx3 = xi.T.reshape(_HIST, _BG, 128).transpose(1, 0, 2)
    out5 = _gather(weight, idx3)
    return out5.transpose(2, 4, 0, 1, 3).reshape(_BATCH, _HIST, _HIDDEN)
